# baseline (speedup 1.0000x reference)
import jax
import jax.numpy as jnp
from jax import lax
from jax.experimental import pallas as pl
from jax.experimental.pallas import tpu as pltpu
from jax.experimental.pallas import tpu_sc as plsc

VOCAB = 1_000_000
D = 64
SEQ = 200
BATCH = 1024
ROWS = BATCH * SEQ
PACKED = VOCAB // 2
HSEQ = SEQ // 2

NC = 2
NS = 16
LANES = 16
NW = NC * NS

SEQ_PER_W = BATCH // NW
SPLITS = ((0, 128), (128, 72))

SCALE = 8.0


def _body(idx_ref, half_ref, tab_ref, pos_ref, out_ref,
          idx_v, half_v, pos_v, buf0, buf1, ob0, ob1, gsems, ssems):
    wid = lax.axis_index("s") * NC + lax.axis_index("c")
    base_seq = wid * SEQ_PER_W
    bufs = (buf0, buf1)
    obufs = (ob0, ob1)

    blk = pl.ds(base_seq * SEQ, SEQ_PER_W * SEQ)
    pltpu.sync_copy(idx_ref.at[blk], idx_v)
    pltpu.sync_copy(half_ref.at[blk], half_v)
    pltpu.sync_copy(pos_ref, pos_v)

    def issue_gather(s, b):
        row0 = s * SEQ
        for off, n in SPLITS:
            pltpu.async_copy(
                tab_ref.at[idx_v.at[pl.ds(row0 + off, n)]],
                bufs[b].at[pl.ds(off, n)],
                gsems.at[b],
            )

    def drain_gather(b):
        pltpu.make_async_copy(
            tab_ref.at[pl.ds(0, SEQ)], bufs[b], gsems.at[b]
        ).wait()

    def drain_store(ob):
        pltpu.make_async_copy(
            out_ref.at[pl.ds(0, 2 * HSEQ)], obufs[ob], ssems.at[ob]
        ).wait()

    def compute(s, b, ob, poff):
        src, dst = bufs[b], obufs[ob]
        row0 = s * SEQ

        @plsc.parallel_loop(0, SEQ, unroll=2)
        def _(r):
            hsel = plsc.load_gather(
                half_v, [jnp.broadcast_to(row0 + r, (LANES,))]
            )
            m = hsel == 1
            p = poff + r // 2
            q = (r % 2) * D
            for c in range(D // LANES):
                lo = src[r, pl.ds(c * LANES, LANES)]
                hi = src[r, pl.ds(D + c * LANES, LANES)]
                dst[p, pl.ds(q + c * LANES, LANES)] = (
                    jnp.where(m, hi, lo) * SCALE
                    + pos_v[r // 2, pl.ds(q + c * LANES, LANES)]
                )

    issue_gather(0, 0)

    @pl.loop(0, SEQ_PER_W, step=4)
    def _(g):
        for j in range(4):
            s = g + j
            b = j % 2
            ob = j // 2

            @pl.when(s + 1 < SEQ_PER_W)
            def _():
                issue_gather(s + 1, 1 - b)

            drain_gather(b)

            if j % 2 == 0:
                @pl.when(s >= 4)
                def _():
                    drain_store(ob)

            compute(s, b, ob, (j % 2) * HSEQ)

            if j % 2 == 1:
                start = pl.multiple_of((base_seq + s - 1) * HSEQ, 8)
                pltpu.async_copy(
                    obufs[ob],
                    out_ref.at[pl.ds(start, 2 * HSEQ)],
                    ssems.at[ob],
                )

    drain_store(0)
    drain_store(1)


@jax.jit
def _embed(idx2, half, tab2, pos2):
    mesh = plsc.VectorSubcoreMesh(
        core_axis_name="c", subcore_axis_name="s", num_cores=NC,
        num_subcores=NS,
    )
    f = pl.kernel(
        _body,
        out_type=jax.ShapeDtypeStruct((ROWS // 2, 2 * D), jnp.float32),
        mesh=mesh,
        scratch_types=[
            pltpu.VMEM((SEQ_PER_W * SEQ,), jnp.int32),
            pltpu.VMEM((SEQ_PER_W * SEQ,), jnp.int32),
            pltpu.VMEM((HSEQ, 2 * D), jnp.float32),
            pltpu.VMEM((SEQ, 2 * D), jnp.float32),
            pltpu.VMEM((SEQ, 2 * D), jnp.float32),
            pltpu.VMEM((SEQ, 2 * D), jnp.float32),
            pltpu.VMEM((SEQ, 2 * D), jnp.float32),
            pltpu.SemaphoreType.DMA((2,)),
            pltpu.SemaphoreType.DMA((2,)),
        ],
        compiler_params=pltpu.CompilerParams(
            use_tc_tiling_on_sc=True, needs_layout_passes=False,
        ),
    )
    return f(idx2, half, tab2, pos2)


def kernel(x, token_table, pos_table):
    x_flat = x.reshape(ROWS).astype(jnp.int32)
    idx2 = x_flat >> 1
    half = x_flat & 1
    tab2 = token_table.reshape(PACKED, 2 * D)
    pos2 = pos_table.reshape(HSEQ, 2 * D)
    out = _embed(idx2, half, tab2, pos2)
    return out.reshape(BATCH, SEQ, D)


# speedup vs baseline: 1.0971x; 1.0971x over previous
import jax
import jax.numpy as jnp
from jax import lax
from jax.experimental import pallas as pl
from jax.experimental.pallas import tpu as pltpu
from jax.experimental.pallas import tpu_sc as plsc

VOCAB = 1_000_000
D = 64
SEQ = 200
BATCH = 1024
ROWS = BATCH * SEQ
HSEQ = SEQ // 2

NC = 2
NS = 16
LANES = 16
NW = NC * NS

SEQ_PER_W = BATCH // NW
SPLITS = ((0, 128), (128, 72))

SCALE = 8.0


def _body(idx_ref, tab_ref, pos_ref, out_ref,
          idx_v, pos_v, buf0, buf1, ob0, ob1, gsems, ssems):
    wid = lax.axis_index("s") * NC + lax.axis_index("c")
    base_seq = wid * SEQ_PER_W
    bufs = (buf0, buf1)
    obufs = (ob0, ob1)

    blk = pl.ds(base_seq * SEQ, SEQ_PER_W * SEQ)
    pltpu.sync_copy(idx_ref.at[blk], idx_v)
    pltpu.sync_copy(pos_ref, pos_v)

    def issue_gather(s, b):
        row0 = s * SEQ
        for off, n in SPLITS:
            pltpu.async_copy(
                tab_ref.at[idx_v.at[pl.ds(row0 + off, n)]],
                bufs[b].at[pl.ds(off, n)],
                gsems.at[b],
            )

    def drain_gather(b):
        pltpu.make_async_copy(
            tab_ref.at[pl.ds(0, SEQ)], bufs[b], gsems.at[b]
        ).wait()

    def drain_store(ob):
        pltpu.make_async_copy(
            out_ref.at[pl.ds(0, 2 * HSEQ)], obufs[ob], ssems.at[ob]
        ).wait()

    def compute(s, b, ob, poff):
        src, dst = bufs[b], obufs[ob]

        @plsc.parallel_loop(0, SEQ, unroll=4)
        def _(r):
            p = poff + r // 2
            q = (r % 2) * D
            for c in range(D // LANES):
                v = src[r, pl.ds(c * LANES, LANES)]
                dst[p, pl.ds(q + c * LANES, LANES)] = (
                    v * SCALE + pos_v[r // 2, pl.ds(q + c * LANES, LANES)]
                )

    issue_gather(0, 0)

    @pl.loop(0, SEQ_PER_W, step=4)
    def _(g):
        for j in range(4):
            s = g + j
            b = j % 2
            ob = j // 2

            @pl.when(s + 1 < SEQ_PER_W)
            def _():
                issue_gather(s + 1, 1 - b)

            drain_gather(b)

            if j % 2 == 0:
                @pl.when(s >= 4)
                def _():
                    drain_store(ob)

            compute(s, b, ob, (j % 2) * HSEQ)

            if j % 2 == 1:
                start = pl.multiple_of((base_seq + s - 1) * HSEQ, 8)
                pltpu.async_copy(
                    obufs[ob],
                    out_ref.at[pl.ds(start, 2 * HSEQ)],
                    ssems.at[ob],
                )

    drain_store(0)
    drain_store(1)


@jax.jit
def _embed(idx, tabp, pos2):
    mesh = plsc.VectorSubcoreMesh(
        core_axis_name="c", subcore_axis_name="s", num_cores=NC,
        num_subcores=NS,
    )
    f = pl.kernel(
        _body,
        out_type=jax.ShapeDtypeStruct((ROWS // 2, 2 * D), jnp.float32),
        mesh=mesh,
        scratch_types=[
            pltpu.VMEM((SEQ_PER_W * SEQ,), jnp.int32),
            pltpu.VMEM((HSEQ, 2 * D), jnp.float32),
            pltpu.VMEM((SEQ, 2 * D), jnp.float32),
            pltpu.VMEM((SEQ, 2 * D), jnp.float32),
            pltpu.VMEM((SEQ, 2 * D), jnp.float32),
            pltpu.VMEM((SEQ, 2 * D), jnp.float32),
            pltpu.SemaphoreType.DMA((2,)),
            pltpu.SemaphoreType.DMA((2,)),
        ],
        compiler_params=pltpu.CompilerParams(
            use_tc_tiling_on_sc=True, needs_layout_passes=False,
        ),
    )
    return f(idx, tabp, pos2)


def kernel(x, token_table, pos_table):
    x_flat = x.reshape(ROWS).astype(jnp.int32)
    tabp = jnp.pad(token_table, ((0, 0), (0, D)))
    pos2 = pos_table.reshape(HSEQ, 2 * D)
    out = _embed(x_flat, tabp, pos2)
    return out.reshape(BATCH, SEQ, D)


# speedup vs baseline: 1.1580x; 1.0555x over previous
import jax
import jax.numpy as jnp
from jax import lax
from jax.experimental import pallas as pl
from jax.experimental.pallas import tpu as pltpu
from jax.experimental.pallas import tpu_sc as plsc

VOCAB = 1_000_000
D = 64
SEQ = 200
BATCH = 1024
ROWS = BATCH * SEQ
HSEQ = SEQ // 2

NC = 2
NS = 16
LANES = 16
NW = NC * NS

SEQ_PER_W = BATCH // NW
SPLITS = ((0, 128), (128, 72))

SCALE = 8.0


def _body(idx_ref, tab_ref, pos_ref, out_ref,
          idx_v, pos_v, buf0, buf1, ob0, ob1, gsems, ssems):
    wid = lax.axis_index("s") * NC + lax.axis_index("c")
    base_seq = wid * SEQ_PER_W
    bufs = (buf0, buf1)
    obufs = (ob0, ob1)

    blk = pl.ds(base_seq * SEQ, SEQ_PER_W * SEQ)
    pltpu.sync_copy(idx_ref.at[blk], idx_v)
    pltpu.sync_copy(pos_ref, pos_v)

    def issue_gather(s, b):
        row0 = s * SEQ
        for off, n in SPLITS:
            pltpu.async_copy(
                tab_ref.at[idx_v.at[pl.ds(row0 + off, n)]],
                bufs[b].at[pl.ds(off, n)],
                gsems.at[b],
            )

    def drain_gather(b):
        pltpu.make_async_copy(
            tab_ref.at[pl.ds(0, SEQ)], bufs[b], gsems.at[b]
        ).wait()

    def drain_store(ob):
        pltpu.make_async_copy(
            out_ref.at[0], obufs[ob], ssems.at[ob]
        ).wait()

    def compute(s, b, ob):
        src, dst = bufs[b], obufs[ob]

        @plsc.parallel_loop(0, SEQ, unroll=4)
        def _(r):
            for c in range(D // LANES):
                v = src[r, pl.ds(c * LANES, LANES)]
                dst[r, pl.ds(c * LANES, LANES)] = (
                    v * SCALE
                    + pos_v[r // 2, pl.ds((r % 2) * D + c * LANES, LANES)]
                )

    issue_gather(0, 0)

    @pl.loop(0, SEQ_PER_W, step=2)
    def _(g):
        for b in range(2):
            s = g + b

            @pl.when(s + 1 < SEQ_PER_W)
            def _():
                issue_gather(s + 1, 1 - b)

            drain_gather(b)

            @pl.when(s >= 2)
            def _():
                drain_store(b)

            compute(s, b, b)
            pltpu.async_copy(
                obufs[b], out_ref.at[base_seq + s], ssems.at[b]
            )

    drain_store(0)
    drain_store(1)


@jax.jit
def _embed(idx, tabp, pos2):
    mesh = plsc.VectorSubcoreMesh(
        core_axis_name="c", subcore_axis_name="s", num_cores=NC,
        num_subcores=NS,
    )
    f = pl.kernel(
        _body,
        out_type=jax.ShapeDtypeStruct((BATCH, SEQ, D), jnp.float32),
        mesh=mesh,
        scratch_types=[
            pltpu.VMEM((SEQ_PER_W * SEQ,), jnp.int32),
            pltpu.VMEM((HSEQ, 2 * D), jnp.float32),
            pltpu.VMEM((SEQ, 2 * D), jnp.float32),
            pltpu.VMEM((SEQ, 2 * D), jnp.float32),
            pltpu.VMEM((SEQ, D), jnp.float32),
            pltpu.VMEM((SEQ, D), jnp.float32),
            pltpu.SemaphoreType.DMA((2,)),
            pltpu.SemaphoreType.DMA((2,)),
        ],
        compiler_params=pltpu.CompilerParams(
            use_tc_tiling_on_sc=True, needs_layout_passes=False,
        ),
    )
    return f(idx, tabp, pos2)


def kernel(x, token_table, pos_table):
    x_flat = x.reshape(ROWS).astype(jnp.int32)
    tabp = jnp.pad(token_table, ((0, 0), (0, D)))
    pos2 = pos_table.reshape(HSEQ, 2 * D)
    return _embed(x_flat, tabp, pos2)
